# baseline (speedup 1.0000x reference)
import functools

import jax
import jax.numpy as jnp
import numpy as np
from jax import lax
from jax.experimental import pallas as pl
from jax.experimental.pallas import tpu as pltpu
from jax.experimental.pallas import tpu_sc as plsc

B = 16384
NUM_NUMERICAL = 13
N_CAT = 26
VOCAB = 100000
D_TOKEN = 32
N_TOK = 1 + NUM_NUMERICAL + N_CAT

CAT_PAD = 32
NUM_PAD = 16
CH = 64
LANES = 16


def _tokenizer_body(x_num_hbm, x_catp_hbm, cls_hbm, w_hbm, bias_hbm,
                    tables_hbm, off_hbm, out_hbm,
                    xcat_v, idx_v, off_v, xnum_v, w_v, b_v, cls_v, out_v,
                    gsem):
    info = plsc.get_sparse_core_info()
    nc, ns = info.num_cores, info.num_subcores
    nw = nc * ns
    rows_per_w = B // nw
    nch = rows_per_w // CH

    wid = lax.axis_index("s") * nc + lax.axis_index("c")

    pltpu.sync_copy(w_hbm, w_v)
    pltpu.sync_copy(bias_hbm, b_v)
    pltpu.sync_copy(cls_hbm, cls_v)
    pltpu.sync_copy(off_hbm, off_v)

    def chunk_body(k, carry):
        base = (wid * nch + k) * CH

        pltpu.sync_copy(x_catp_hbm.at[pl.ds(base * CAT_PAD, CH * CAT_PAD)],
                        xcat_v)
        pltpu.sync_copy(x_num_hbm.at[pl.ds(base * NUM_PAD, CH * NUM_PAD)],
                        xnum_v)

        def idx_body(i, _):
            s = pl.ds(i * LANES, LANES)
            idx_v[s] = xcat_v[s] + off_v[s]
            return 0

        lax.fori_loop(0, CH * CAT_PAD // LANES, idx_body, 0)

        def fire_body(bi, _):
            o = pl.multiple_of(bi * CAT_PAD, CAT_PAD)
            pltpu.async_copy(tables_hbm.at[idx_v.at[pl.ds(o, N_CAT)]],
                             out_v.at[bi, pl.ds(1 + NUM_NUMERICAL, N_CAT)],
                             gsem)
            return 0

        lax.fori_loop(0, CH, fire_body, 0)

        cls0 = cls_v[pl.ds(0, LANES)]
        cls1 = cls_v[pl.ds(LANES, LANES)]

        def num_body(bi, _):
            out_v[bi, 0, pl.ds(0, LANES)] = cls0
            out_v[bi, 0, pl.ds(LANES, LANES)] = cls1
            xrow = xnum_v[pl.ds(bi * NUM_PAD, NUM_PAD)]
            for j in range(NUM_NUMERICAL):
                xs = xrow[j]
                out_v[bi, 1 + j, pl.ds(0, LANES)] = (
                    xs * w_v[j, pl.ds(0, LANES)] + b_v[j, pl.ds(0, LANES)])
                out_v[bi, 1 + j, pl.ds(LANES, LANES)] = (
                    xs * w_v[j, pl.ds(LANES, LANES)] + b_v[j, pl.ds(LANES, LANES)])
            return 0

        lax.fori_loop(0, CH, num_body, 0)

        def drain_body(bi, _):
            o = pl.multiple_of(bi * CAT_PAD, CAT_PAD)
            pltpu.make_async_copy(
                tables_hbm.at[idx_v.at[pl.ds(o, N_CAT)]],
                out_v.at[bi, pl.ds(1 + NUM_NUMERICAL, N_CAT)],
                gsem).wait()
            return 0

        lax.fori_loop(0, CH, drain_body, 0)

        pltpu.sync_copy(out_v, out_hbm.at[pl.ds(base, CH)])
        return 0

    lax.fori_loop(0, nch, chunk_body, 0)


@functools.partial(jax.jit, static_argnames=())
def kernel(x_num, x_cat, cls_token, num_weights, num_biases, cat_tables):
    tables_flat = cat_tables.reshape(N_CAT * VOCAB, D_TOKEN)
    x_nump = jnp.concatenate(
        [x_num, jnp.zeros((B, NUM_PAD - NUM_NUMERICAL), jnp.float32)], axis=1
    ).reshape(B * NUM_PAD)
    x_cat32 = x_cat.astype(jnp.int32)
    x_catp = jnp.concatenate(
        [x_cat32, jnp.zeros((B, CAT_PAD - N_CAT), jnp.int32)], axis=1
    ).reshape(B * CAT_PAD)
    cls_flat = cls_token.reshape(D_TOKEN)

    off_np = np.zeros((CH, CAT_PAD), np.int32)
    off_np[:, :N_CAT] = np.arange(N_CAT, dtype=np.int32)[None, :] * VOCAB
    offsets = jnp.asarray(off_np.reshape(CH * CAT_PAD))

    mesh = plsc.VectorSubcoreMesh(core_axis_name="c", subcore_axis_name="s")
    run = pl.kernel(
        _tokenizer_body,
        out_type=jax.ShapeDtypeStruct((B, N_TOK, D_TOKEN), jnp.float32),
        mesh=mesh,
        compiler_params=pltpu.CompilerParams(use_tc_tiling_on_sc=False),
        scratch_types=[
            pltpu.VMEM((CH * CAT_PAD,), jnp.int32),
            pltpu.VMEM((CH * CAT_PAD,), jnp.int32),
            pltpu.VMEM((CH * CAT_PAD,), jnp.int32),
            pltpu.VMEM((CH * NUM_PAD,), jnp.float32),
            pltpu.VMEM((NUM_NUMERICAL, D_TOKEN), jnp.float32),
            pltpu.VMEM((NUM_NUMERICAL, D_TOKEN), jnp.float32),
            pltpu.VMEM((D_TOKEN,), jnp.float32),
            pltpu.VMEM((CH, N_TOK, D_TOKEN), jnp.float32),
            pltpu.SemaphoreType.DMA,
        ],
    )
    return run(x_nump, x_catp, cls_flat, num_weights, num_biases,
               tables_flat, offsets)


# speedup vs baseline: 1.0029x; 1.0029x over previous
import functools

import jax
import jax.numpy as jnp
import numpy as np
from jax import lax
from jax.experimental import pallas as pl
from jax.experimental.pallas import tpu as pltpu
from jax.experimental.pallas import tpu_sc as plsc

B = 16384
NUM_NUMERICAL = 13
N_CAT = 26
VOCAB = 100000
D_TOKEN = 32
N_TOK = 1 + NUM_NUMERICAL + N_CAT
N_NC = 1 + NUM_NUMERICAL

NUM_PAD = 16
CH = 64
LANES = 16


def _tokenizer_body(x_num_hbm, x_cat_hbm, cls_hbm, w_hbm, bias_hbm,
                    tables_hbm, off_hbm, cpat_hbm, npat_hbm, out_hbm,
                    xcat_v, idx_v, off_v, cpat_v, npat_v, cdidx_v, ndidx_v,
                    xnum_v, w_v, b_v, cls_v, cat_v, numcls_v, gsem, ssem):
    info = plsc.get_sparse_core_info()
    nc, ns = info.num_cores, info.num_subcores
    nw = nc * ns
    rows_per_w = B // nw
    nch = rows_per_w // CH
    ids_per_ch = CH * N_CAT
    nc_per_ch = CH * N_NC

    wid = lax.axis_index("s") * nc + lax.axis_index("c")

    pltpu.sync_copy(w_hbm, w_v)
    pltpu.sync_copy(bias_hbm, b_v)
    pltpu.sync_copy(cls_hbm, cls_v)
    pltpu.sync_copy(off_hbm, off_v)
    pltpu.sync_copy(cpat_hbm, cpat_v)
    pltpu.sync_copy(npat_hbm, npat_v)

    def chunk_body(k, carry):
        base = (wid * nch + k) * CH

        pltpu.sync_copy(x_cat_hbm.at[pl.ds(base * N_CAT, ids_per_ch)], xcat_v)
        pltpu.sync_copy(x_num_hbm.at[pl.ds(base * NUM_PAD, CH * NUM_PAD)],
                        xnum_v)

        obase = base * N_TOK

        def idx_body(i, _):
            s = pl.ds(i * LANES, LANES)
            idx_v[s] = xcat_v[s] + off_v[s]
            return 0

        def cd_body(i, _):
            s = pl.ds(i * LANES, LANES)
            cdidx_v[s] = cpat_v[s] + obase
            return 0

        def nd_body(i, _):
            s = pl.ds(i * LANES, LANES)
            ndidx_v[s] = npat_v[s] + obase
            return 0

        lax.fori_loop(0, ids_per_ch // LANES, idx_body, 0)
        lax.fori_loop(0, ids_per_ch // LANES, cd_body, 0)
        lax.fori_loop(0, nc_per_ch // LANES, nd_body, 0)

        pltpu.async_copy(tables_hbm.at[idx_v], cat_v, gsem)

        cls0 = cls_v[pl.ds(0, LANES)]
        cls1 = cls_v[pl.ds(LANES, LANES)]

        def num_body(bi, _):
            r = bi * N_NC
            numcls_v[r, pl.ds(0, LANES)] = cls0
            numcls_v[r, pl.ds(LANES, LANES)] = cls1
            xrow = xnum_v[pl.ds(bi * NUM_PAD, NUM_PAD)]
            for j in range(NUM_NUMERICAL):
                xs = xrow[j]
                numcls_v[r + 1 + j, pl.ds(0, LANES)] = (
                    xs * w_v[j, pl.ds(0, LANES)] + b_v[j, pl.ds(0, LANES)])
                numcls_v[r + 1 + j, pl.ds(LANES, LANES)] = (
                    xs * w_v[j, pl.ds(LANES, LANES)] + b_v[j, pl.ds(LANES, LANES)])
            return 0

        lax.fori_loop(0, CH, num_body, 0)

        pltpu.async_copy(numcls_v, out_hbm.at[ndidx_v], ssem)

        pltpu.make_async_copy(tables_hbm.at[idx_v], cat_v, gsem).wait()
        pltpu.async_copy(cat_v, out_hbm.at[cdidx_v], ssem)

        pltpu.make_async_copy(numcls_v, out_hbm.at[ndidx_v], ssem).wait()
        pltpu.make_async_copy(cat_v, out_hbm.at[cdidx_v], ssem).wait()
        return 0

    lax.fori_loop(0, nch, chunk_body, 0)


@functools.partial(jax.jit, static_argnames=())
def kernel(x_num, x_cat, cls_token, num_weights, num_biases, cat_tables):
    tables_flat = cat_tables.reshape(N_CAT * VOCAB, D_TOKEN)
    x_nump = jnp.concatenate(
        [x_num, jnp.zeros((B, NUM_PAD - NUM_NUMERICAL), jnp.float32)], axis=1
    ).reshape(B * NUM_PAD)
    x_cat_flat = x_cat.astype(jnp.int32).reshape(B * N_CAT)
    cls_flat = cls_token.reshape(D_TOKEN)

    p = np.arange(CH * N_CAT, dtype=np.int32)
    off_np = np.tile(np.arange(N_CAT, dtype=np.int32) * VOCAB, CH)
    cpat_np = (p // N_CAT) * N_TOK + N_NC + (p % N_CAT)
    q = np.arange(CH * N_NC, dtype=np.int32)
    npat_np = (q // N_NC) * N_TOK + (q % N_NC)

    mesh = plsc.VectorSubcoreMesh(core_axis_name="c", subcore_axis_name="s")
    run = pl.kernel(
        _tokenizer_body,
        out_type=jax.ShapeDtypeStruct((B * N_TOK, D_TOKEN), jnp.float32),
        mesh=mesh,
        compiler_params=pltpu.CompilerParams(use_tc_tiling_on_sc=False),
        scratch_types=[
            pltpu.VMEM((CH * N_CAT,), jnp.int32),
            pltpu.VMEM((CH * N_CAT,), jnp.int32),
            pltpu.VMEM((CH * N_CAT,), jnp.int32),
            pltpu.VMEM((CH * N_CAT,), jnp.int32),
            pltpu.VMEM((CH * N_NC,), jnp.int32),
            pltpu.VMEM((CH * N_CAT,), jnp.int32),
            pltpu.VMEM((CH * N_NC,), jnp.int32),
            pltpu.VMEM((CH * NUM_PAD,), jnp.float32),
            pltpu.VMEM((NUM_NUMERICAL, D_TOKEN), jnp.float32),
            pltpu.VMEM((NUM_NUMERICAL, D_TOKEN), jnp.float32),
            pltpu.VMEM((D_TOKEN,), jnp.float32),
            pltpu.VMEM((CH * N_CAT, D_TOKEN), jnp.float32),
            pltpu.VMEM((CH * N_NC, D_TOKEN), jnp.float32),
            pltpu.SemaphoreType.DMA,
            pltpu.SemaphoreType.DMA,
        ],
    )
    out_flat = run(x_nump, x_cat_flat, cls_flat, num_weights, num_biases,
                   tables_flat, jnp.asarray(off_np), jnp.asarray(cpat_np),
                   jnp.asarray(npat_np))
    return out_flat.reshape(B, N_TOK, D_TOKEN)
